# Initial kernel scaffold; baseline (speedup 1.0000x reference)
import jax
import jax.numpy as jnp
from jax.experimental import pallas as pl


def kernel(x, edge_index, W1, b1, W2, b2):
    raise NotImplementedError("write your pallas kernel here")


# speedup vs baseline: 24.6488x; 24.6488x over previous
import functools
import jax
import jax.numpy as jnp
from jax import lax
from jax.experimental import pallas as pl
from jax.experimental.pallas import tpu as pltpu
from jax.experimental.pallas import tpu_sc as plsc

NC, NS, LANES = 2, 16, 16
NW = NC * NS


def _mesh():
    return plsc.VectorSubcoreMesh(core_axis_name="c", subcore_axis_name="s",
                                  num_cores=NC, num_subcores=NS)


def _make_deg_kernel(E, NP):
    EPW = E // NW

    @functools.partial(
        pl.kernel,
        out_type=jax.ShapeDtypeStruct((NW, NP), jnp.float32),
        mesh=_mesh(),
        scratch_types=[
            pltpu.VMEM((NP,), jnp.float32),
            pltpu.VMEM((EPW,), jnp.int32),
        ],
        compiler_params=pltpu.CompilerParams(needs_layout_passes=False),
    )
    def deg_kernel(dst_hbm, out_hbm, hist_v, idx_v):
        c = lax.axis_index("c")
        s = lax.axis_index("s")
        wid = c * NS + s

        def zero_body(i, carry):
            hist_v[pl.ds(i * LANES, LANES)] = jnp.zeros((LANES,), jnp.float32)
            return carry
        lax.fori_loop(0, NP // LANES, zero_body, 0)

        pltpu.sync_copy(dst_hbm.at[pl.ds(wid * EPW, EPW)], idx_v)

        ones = jnp.full((LANES,), 1.0, jnp.float32)

        def body(i, carry):
            idx = idx_v[pl.ds(i * LANES, LANES)]
            plsc.addupdate_scatter(hist_v, [idx], ones)
            return carry
        lax.fori_loop(0, EPW // LANES, body, 0)

        pltpu.sync_copy(hist_v, out_hbm.at[wid])

    return deg_kernel


def _make_scatter_kernel(NPAD, F, NB, B):
    SLAB = NPAD // NS
    ZR = 64

    @functools.partial(
        pl.kernel,
        out_type=jax.ShapeDtypeStruct((NC, NPAD, F), jnp.float32),
        mesh=_mesh(),
        scratch_types=[
            pltpu.VMEM_SHARED((NPAD, F), jnp.float32),
            pltpu.VMEM((NB, B), jnp.int32),
            pltpu.VMEM((NB, B), jnp.int32),
            pltpu.VMEM((ZR, F), jnp.float32),
            pltpu.VMEM((B, F), jnp.float32),
            pltpu.SemaphoreType.DMA,
        ],
        compiler_params=pltpu.CompilerParams(needs_layout_passes=False),
    )
    def scatter_kernel(y_hbm, src_hbm, dst_hbm, out_hbm,
                       acc_sh, src_v, dst_v, zero_v, rows_v, sem):
        c = lax.axis_index("c")
        s = lax.axis_index("s")
        wid = c * NS + s

        def zb(i, carry):
            r = i // (F // LANES)
            col = i % (F // LANES)
            zero_v[r, pl.ds(col * LANES, LANES)] = jnp.zeros((LANES,), jnp.float32)
            return carry
        lax.fori_loop(0, ZR * (F // LANES), zb, 0)
        for k in range(SLAB // ZR):
            pltpu.sync_copy(zero_v, acc_sh.at[pl.ds(s * SLAB + k * ZR, ZR)])
        plsc.subcore_barrier()

        pltpu.sync_copy(src_hbm.at[wid], src_v)
        pltpu.sync_copy(dst_hbm.at[wid], dst_v)

        def body(j, carry):
            pltpu.async_copy(y_hbm.at[src_v.at[j]], rows_v, sem).wait()
            pltpu.sync_copy(rows_v, acc_sh.at[dst_v.at[j]], add=True)
            return carry
        lax.fori_loop(0, NB, body, 0)

        plsc.subcore_barrier()
        pltpu.sync_copy(acc_sh.at[pl.ds(s * SLAB, SLAB)],
                        out_hbm.at[c, pl.ds(s * SLAB, SLAB)])

    return scatter_kernel


def _dinv_col(degp_blk):
    d = jnp.transpose(degp_blk)
    deg = jnp.sum(d, axis=1, keepdims=True) + 1.0
    return lax.rsqrt(deg)


def _y1_body(degp_ref, x_ref, y_ref):
    dinv = _dinv_col(degp_ref[...])
    y_ref[...] = x_ref[...] * dinv


def _mid_body(degp_ref, sp_ref, y1_ref, w1_ref, b1_ref, w2_ref, y2_ref):
    dinv = _dinv_col(degp_ref[...])
    px = dinv * (sp_ref[0] + sp_ref[1] + y1_ref[...])
    h1 = jnp.dot(px, w1_ref[...], preferred_element_type=jnp.float32,
                 precision=lax.Precision.HIGHEST)
    h1 = jnp.maximum(h1 + b1_ref[...], 0.0)
    h2 = jnp.dot(h1, w2_ref[...], preferred_element_type=jnp.float32,
                 precision=lax.Precision.HIGHEST)
    y2_ref[...] = h2 * dinv


def _out_body(degp_ref, sp_ref, y2_ref, b2_ref, o_ref):
    dinv = _dinv_col(degp_ref[...])
    o_ref[...] = dinv * (sp_ref[0] + sp_ref[1] + y2_ref[...]) + b2_ref[...]


@jax.jit
def kernel(x, edge_index, W1, b1, W2, b2):
    N, F = x.shape
    E = edge_index.shape[1]
    NP = 10240
    BN = 1024
    EPW = E // NW
    B = 125
    NB = EPW // B

    src = edge_index[0].astype(jnp.int32)
    dst = edge_index[1].astype(jnp.int32)
    src3 = src.reshape(NW, NB, B)
    dst3 = dst.reshape(NW, NB, B)

    degp = _make_deg_kernel(E, NP)(dst)

    grid = (NP // BN,)
    degp_spec = pl.BlockSpec((NW, BN), lambda i: (0, i))
    row_spec = pl.BlockSpec((BN, F), lambda i: (i, 0))
    sp_spec = pl.BlockSpec((NC, BN, F), lambda i: (0, i, 0))

    y1 = pl.pallas_call(
        _y1_body,
        grid=grid,
        in_specs=[degp_spec, row_spec],
        out_specs=row_spec,
        out_shape=jax.ShapeDtypeStruct((N, F), jnp.float32),
    )(degp, x)

    scat = _make_scatter_kernel(NP, F, NB, B)
    s1p = scat(y1, src3, dst3)

    y2 = pl.pallas_call(
        _mid_body,
        grid=grid,
        in_specs=[degp_spec, sp_spec, row_spec,
                  pl.BlockSpec((F, 2 * F), lambda i: (0, 0)),
                  pl.BlockSpec((1, 2 * F), lambda i: (0, 0)),
                  pl.BlockSpec((2 * F, F), lambda i: (0, 0))],
        out_specs=row_spec,
        out_shape=jax.ShapeDtypeStruct((N, F), jnp.float32),
    )(degp, s1p, y1, W1, b1.reshape(1, -1), W2)

    s2p = scat(y2, src3, dst3)

    out = pl.pallas_call(
        _out_body,
        grid=grid,
        in_specs=[degp_spec, sp_spec, row_spec,
                  pl.BlockSpec((1, F), lambda i: (0, 0))],
        out_specs=row_spec,
        out_shape=jax.ShapeDtypeStruct((N, F), jnp.float32),
    )(degp, s2p, y2, b2.reshape(1, -1))

    return out
